# baseline (speedup 1.0000x reference)
import functools

import jax
import jax.numpy as jnp
from jax import lax
from jax.experimental import pallas as pl
from jax.experimental.pallas import tpu as pltpu
from jax.experimental.pallas import tpu_sc as plsc

E = 16384
D = 64
L = 16
NC = 2
NS = 16
NW = NC * NS
EPW = E // NW
GROUPS = EPW // L


def _mf_body(usr_table, itm_table, usr_nid, itm_nid, edges, out,
             eidx_u, eidx_i, nid_u, nid_i, rows_u, rows_i, score, sem):
    wid = lax.axis_index("s") * NC + lax.axis_index("c")
    base = wid * EPW

    pltpu.sync_copy(edges.at[0, pl.ds(base, EPW)], eidx_u)
    pltpu.sync_copy(edges.at[1, pl.ds(base, EPW)], eidx_i)

    cu = pltpu.async_copy(usr_nid.at[eidx_u], nid_u, sem)
    ci = pltpu.async_copy(itm_nid.at[eidx_i], nid_i, sem)
    cu.wait()
    ci.wait()

    gu = pltpu.async_copy(usr_table.at[nid_u], rows_u, sem)
    gi = pltpu.async_copy(itm_table.at[nid_i], rows_i, sem)
    gu.wait()
    gi.wait()

    lanes = lax.iota(jnp.int32, L)

    def group_body(g, _):
        rowv = g * L + lanes

        def dim_body(d, acc):
            dd = jnp.full((L,), d, jnp.int32)
            u = plsc.load_gather(rows_u, [rowv, dd])
            v = plsc.load_gather(rows_i, [rowv, dd])
            return acc + u * v

        acc = lax.fori_loop(0, D, dim_body, jnp.zeros((L,), jnp.float32),
                            unroll=8)
        score[pl.ds(g * L, L)] = acc
        return 0

    lax.fori_loop(0, GROUPS, group_body, 0)

    pltpu.sync_copy(score, out.at[pl.ds(base, EPW)])


@jax.jit
def _mf_sc(usr_table, itm_table, usr_n_id, itm_n_id, edge_label_index):
    mesh = plsc.VectorSubcoreMesh(core_axis_name="c", subcore_axis_name="s")
    return pl.kernel(
        _mf_body,
        mesh=mesh,
        compiler_params=pltpu.CompilerParams(
            needs_layout_passes=False, use_tc_tiling_on_sc=False),
        out_type=jax.ShapeDtypeStruct((E,), jnp.float32),
        scratch_types=[
            pltpu.VMEM((EPW,), jnp.int32),
            pltpu.VMEM((EPW,), jnp.int32),
            pltpu.VMEM((EPW,), jnp.int32),
            pltpu.VMEM((EPW,), jnp.int32),
            pltpu.VMEM((EPW, D), jnp.float32),
            pltpu.VMEM((EPW, D), jnp.float32),
            pltpu.VMEM((EPW,), jnp.float32),
            pltpu.SemaphoreType.DMA,
        ],
    )(usr_table, itm_table, usr_n_id, itm_n_id, edge_label_index)


def kernel(usr_table, itm_table, usr_n_id, itm_n_id, edge_label_index):
    return _mf_sc(usr_table, itm_table, usr_n_id, itm_n_id, edge_label_index)
